# baseline (speedup 1.0000x reference)
import math

import jax
import jax.numpy as jnp
from jax import lax
from jax.experimental import pallas as pl
from jax.experimental.pallas import tpu as pltpu
from jax.experimental.pallas import tpu_sc as plsc

_TB = 512
_BT = 2560
_R_CH = 16
_LN2 = math.log(2.0)


def _mog_logprob_kernel(x_ref, mean_ref, scale_ref, wl_ref, out_ref):
    x = x_ref[...]
    wl = wl_ref[...]
    log_w = wl - jax.nn.logsumexp(wl, axis=-1, keepdims=True)

    k_tot = mean_ref.shape[0]
    half_log_2pi = 0.5 * math.log(2.0 * math.pi)

    x2 = x * x
    coef = []
    for k in range(k_tot):
        sc = jax.nn.softplus(scale_ref[k, :])[None, :]
        mu = mean_ref[k, :][None, :]
        q = -0.5 / (sc * sc)
        b = -2.0 * q * mu
        a = q * mu * mu - jnp.log(sc) - half_log_2pi + log_w[0:1, k : k + 1]
        coef.append((a, b, q))

    mf = None
    for a, b, q in coef:
        v = a + b * x + q * x2
        mf = v if mf is None else jnp.maximum(mf, v)
    s = None
    for a, b, q in coef:
        e = jnp.exp(a + b * x + q * x2 - mf)
        s = e if s is None else s + e
    out_ref[...] = mf + jnp.log(s)


def _coef_kernel(mean_ref, scale_ref, wl_ref, coef_ref):
    wl = wl_ref[...]
    log_w = wl - jax.nn.logsumexp(wl, axis=-1, keepdims=True)
    k_tot = mean_ref.shape[0]
    half_log_2pi = 0.5 * math.log(2.0 * math.pi)
    rows_a, rows_b, rows_q = [], [], []
    for k in range(k_tot):
        sc = jax.nn.softplus(scale_ref[k, :])[None, :]
        mu = mean_ref[k, :][None, :]
        q = -0.5 / (sc * sc)
        b = -2.0 * q * mu
        a = q * mu * mu - jnp.log(sc) - half_log_2pi + log_w[0:1, k : k + 1]
        rows_a.append(a)
        rows_b.append(b)
        rows_q.append(q)
    coef_ref[...] = jnp.concatenate(rows_a + rows_b + rows_q, axis=0)


def _log_1plus(s):
    bits = lax.bitcast_convert_type(s, jnp.int32)
    e = jnp.right_shift(bits, 23) - 127
    fbits = jnp.bitwise_or(
        jnp.bitwise_and(bits, jnp.int32(0x7FFFFF)), jnp.int32(0x3F800000)
    )
    f = lax.bitcast_convert_type(fbits, jnp.float32)
    t = (f - 1.0) / (f + 1.0)
    t2 = t * t
    p = 1.0 + t2 * (
        (1.0 / 3.0) + t2 * ((1.0 / 5.0) + t2 * ((1.0 / 7.0) + t2 * (1.0 / 9.0)))
    )
    return e.astype(jnp.float32) * _LN2 + 2.0 * t * p


def _make_sc_kernel(b_sc, z, k_tot):
    nw = 32
    rows_per_tec = b_sc // nw
    n_chunks = rows_per_tec // _R_CH
    mesh = plsc.VectorSubcoreMesh(core_axis_name="c", subcore_axis_name="s")

    def sc_kernel(coef_hbm, x_hbm, out_hbm, coef_v, x_v, out_v, sem):
        wid = lax.axis_index("s") * 2 + lax.axis_index("c")
        base = wid * rows_per_tec
        pltpu.sync_copy(coef_hbm, coef_v)
        for ch in range(n_chunks):
            cb = base + ch * _R_CH
            pltpu.sync_copy(x_hbm.at[pl.ds(cb, _R_CH)], x_v)

            def g_body(g, carry):
                za = g * 16
                cvecs = []
                for k in range(k_tot):
                    a_k = coef_v[k, pl.ds(za, 16)]
                    b_k = coef_v[k_tot + k, pl.ds(za, 16)]
                    q_k = coef_v[2 * k_tot + k, pl.ds(za, 16)]
                    cvecs.append((a_k, b_k, q_k))
                for r in range(_R_CH):
                    xv = x_v[r, pl.ds(za, 16)]
                    x2 = xv * xv
                    vs = [a + b * xv + q * x2 for (a, b, q) in cvecs]
                    m = vs[0]
                    for v in vs[1:]:
                        m = jnp.maximum(m, v)
                    s = jnp.exp(vs[0] - m)
                    for v in vs[1:]:
                        s = s + jnp.exp(v - m)
                    out_v[r, pl.ds(za, 16)] = m + _log_1plus(s)
                return carry

            lax.fori_loop(0, z // 16, g_body, 0)
            pltpu.sync_copy(out_v, out_hbm.at[pl.ds(cb, _R_CH)])

    return pl.kernel(
        sc_kernel,
        mesh=mesh,
        out_type=jax.ShapeDtypeStruct((b_sc, z), jnp.float32),
        scratch_types=[
            pltpu.VMEM((3 * k_tot, z), jnp.float32),
            pltpu.VMEM((_R_CH, z), jnp.float32),
            pltpu.VMEM((_R_CH, z), jnp.float32),
            pltpu.SemaphoreType.DMA,
        ],
    )


def kernel(x, mean_list, scale_list, weight_logits):
    b, z = x.shape
    k = mean_list.shape[-1]
    mean_t = mean_list[0].T
    scale_t = scale_list[0].T

    coef = pl.pallas_call(
        _coef_kernel,
        grid=(1,),
        in_specs=[
            pl.BlockSpec((k, z), lambda i: (0, 0)),
            pl.BlockSpec((k, z), lambda i: (0, 0)),
            pl.BlockSpec((1, k), lambda i: (0, 0)),
        ],
        out_specs=pl.BlockSpec((3 * k, z), lambda i: (0, 0)),
        out_shape=jax.ShapeDtypeStruct((3 * k, z), jnp.float32),
    )(mean_t, scale_t, weight_logits)

    out_sc = _make_sc_kernel(b - _BT, z, k)(coef, x[_BT:])

    out_tc = pl.pallas_call(
        _mog_logprob_kernel,
        grid=(_BT // _TB,),
        in_specs=[
            pl.BlockSpec((_TB, z), lambda i: (i, 0)),
            pl.BlockSpec((k, z), lambda i: (0, 0)),
            pl.BlockSpec((k, z), lambda i: (0, 0)),
            pl.BlockSpec((1, k), lambda i: (0, 0)),
        ],
        out_specs=pl.BlockSpec((_TB, z), lambda i: (i, 0)),
        out_shape=jax.ShapeDtypeStruct((_BT, z), x.dtype),
    )(x[:_BT], mean_t, scale_t, weight_logits)

    return jnp.concatenate([out_tc, out_sc], axis=0)


# speedup vs baseline: 1.4376x; 1.4376x over previous
import math

import jax
import jax.numpy as jnp
from jax import lax
from jax.experimental import pallas as pl
from jax.experimental.pallas import tpu as pltpu
from jax.experimental.pallas import tpu_sc as plsc

_TB = 512
_BT = 3072
_R_CH = 16
_LN2 = math.log(2.0)


def _mog_logprob_kernel(x_ref, mean_ref, scale_ref, wl_ref, out_ref):
    x = x_ref[...]
    wl = wl_ref[...]
    log_w = wl - jax.nn.logsumexp(wl, axis=-1, keepdims=True)

    k_tot = mean_ref.shape[0]
    half_log_2pi = 0.5 * math.log(2.0 * math.pi)

    x2 = x * x
    coef = []
    for k in range(k_tot):
        sc = jax.nn.softplus(scale_ref[k, :])[None, :]
        mu = mean_ref[k, :][None, :]
        q = -0.5 / (sc * sc)
        b = -2.0 * q * mu
        a = q * mu * mu - jnp.log(sc) - half_log_2pi + log_w[0:1, k : k + 1]
        coef.append((a, b, q))

    mf = None
    for a, b, q in coef:
        v = a + b * x + q * x2
        mf = v if mf is None else jnp.maximum(mf, v)
    s = None
    for a, b, q in coef:
        e = jnp.exp(a + b * x + q * x2 - mf)
        s = e if s is None else s + e
    out_ref[...] = mf + jnp.log(s)


def _coef_kernel(mean_ref, scale_ref, wl_ref, coef_ref):
    wl = wl_ref[...]
    log_w = wl - jax.nn.logsumexp(wl, axis=-1, keepdims=True)
    k_tot = mean_ref.shape[0]
    half_log_2pi = 0.5 * math.log(2.0 * math.pi)
    rows_a, rows_b, rows_q = [], [], []
    for k in range(k_tot):
        sc = jax.nn.softplus(scale_ref[k, :])[None, :]
        mu = mean_ref[k, :][None, :]
        q = -0.5 / (sc * sc)
        b = -2.0 * q * mu
        a = q * mu * mu - jnp.log(sc) - half_log_2pi + log_w[0:1, k : k + 1]
        rows_a.append(a)
        rows_b.append(b)
        rows_q.append(q)
    coef_ref[...] = jnp.concatenate(rows_a + rows_b + rows_q, axis=0)


def _log_1plus(s):
    bits = lax.bitcast_convert_type(s, jnp.int32)
    e = jnp.right_shift(bits, 23) - 127
    fbits = jnp.bitwise_or(
        jnp.bitwise_and(bits, jnp.int32(0x7FFFFF)), jnp.int32(0x3F800000)
    )
    f = lax.bitcast_convert_type(fbits, jnp.float32)
    t = (f - 1.0) / (f + 1.0)
    t2 = t * t
    p = 1.0 + t2 * (
        (1.0 / 3.0) + t2 * ((1.0 / 5.0) + t2 * ((1.0 / 7.0) + t2 * (1.0 / 9.0)))
    )
    return e.astype(jnp.float32) * _LN2 + 2.0 * t * p


def _make_sc_kernel(b_sc, z, k_tot):
    nw = 32
    rows_per_tec = b_sc // nw
    n_chunks = rows_per_tec // _R_CH
    mesh = plsc.VectorSubcoreMesh(core_axis_name="c", subcore_axis_name="s")

    def sc_kernel(coef_hbm, x_hbm, out_hbm, coef_v, x_v, out_v, sem):
        wid = lax.axis_index("s") * 2 + lax.axis_index("c")
        base = wid * rows_per_tec
        pltpu.sync_copy(coef_hbm, coef_v)
        for ch in range(n_chunks):
            cb = base + ch * _R_CH
            pltpu.sync_copy(x_hbm.at[pl.ds(cb, _R_CH)], x_v)

            def g_body(g, carry):
                za = g * 16
                cvecs = []
                for k in range(k_tot):
                    a_k = coef_v[k, pl.ds(za, 16)]
                    b_k = coef_v[k_tot + k, pl.ds(za, 16)]
                    q_k = coef_v[2 * k_tot + k, pl.ds(za, 16)]
                    cvecs.append((a_k, b_k, q_k))
                for r in range(_R_CH):
                    xv = x_v[r, pl.ds(za, 16)]
                    x2 = xv * xv
                    vs = [a + b * xv + q * x2 for (a, b, q) in cvecs]
                    m = vs[0]
                    for v in vs[1:]:
                        m = jnp.maximum(m, v)
                    s = jnp.exp(vs[0] - m)
                    for v in vs[1:]:
                        s = s + jnp.exp(v - m)
                    out_v[r, pl.ds(za, 16)] = m + _log_1plus(s)
                return carry

            lax.fori_loop(0, z // 16, g_body, 0)
            pltpu.sync_copy(out_v, out_hbm.at[pl.ds(cb, _R_CH)])

    return pl.kernel(
        sc_kernel,
        mesh=mesh,
        out_type=jax.ShapeDtypeStruct((b_sc, z), jnp.float32),
        scratch_types=[
            pltpu.VMEM((3 * k_tot, z), jnp.float32),
            pltpu.VMEM((_R_CH, z), jnp.float32),
            pltpu.VMEM((_R_CH, z), jnp.float32),
            pltpu.SemaphoreType.DMA,
        ],
    )


def kernel(x, mean_list, scale_list, weight_logits):
    b, z = x.shape
    k = mean_list.shape[-1]
    mean_t = mean_list[0].T
    scale_t = scale_list[0].T

    coef = pl.pallas_call(
        _coef_kernel,
        grid=(1,),
        in_specs=[
            pl.BlockSpec((k, z), lambda i: (0, 0)),
            pl.BlockSpec((k, z), lambda i: (0, 0)),
            pl.BlockSpec((1, k), lambda i: (0, 0)),
        ],
        out_specs=pl.BlockSpec((3 * k, z), lambda i: (0, 0)),
        out_shape=jax.ShapeDtypeStruct((3 * k, z), jnp.float32),
    )(mean_t, scale_t, weight_logits)

    out_sc = _make_sc_kernel(b - _BT, z, k)(coef, x[_BT:])

    out_tc = pl.pallas_call(
        _mog_logprob_kernel,
        grid=(_BT // _TB,),
        in_specs=[
            pl.BlockSpec((_TB, z), lambda i: (i, 0)),
            pl.BlockSpec((k, z), lambda i: (0, 0)),
            pl.BlockSpec((k, z), lambda i: (0, 0)),
            pl.BlockSpec((1, k), lambda i: (0, 0)),
        ],
        out_specs=pl.BlockSpec((_TB, z), lambda i: (i, 0)),
        out_shape=jax.ShapeDtypeStruct((b, z), x.dtype),
    )(x[:_BT], mean_t, scale_t, weight_logits)

    return jax.lax.dynamic_update_slice(out_tc, out_sc, (_BT, 0))


# speedup vs baseline: 1.5989x; 1.1122x over previous
import math

import jax
import jax.numpy as jnp
from jax.experimental import pallas as pl

_TB = 512


def _mog_logprob_kernel(x_ref, mean_ref, scale_ref, wl_ref, out_ref):
    x = x_ref[...]
    wl = wl_ref[...]
    log_w = wl - jax.nn.logsumexp(wl, axis=-1, keepdims=True)

    k_tot = mean_ref.shape[0]
    half_log_2pi = 0.5 * math.log(2.0 * math.pi)

    x2 = x * x
    coef = []
    for k in range(k_tot):
        sc = jax.nn.softplus(scale_ref[k, :])[None, :]
        mu = mean_ref[k, :][None, :]
        q = -0.5 / (sc * sc)
        b = -2.0 * q * mu
        a = q * mu * mu - jnp.log(sc) - half_log_2pi + log_w[0:1, k : k + 1]
        coef.append((a, b, q))

    mf = None
    for a, b, q in coef:
        v = a + b * x + q * x2
        mf = v if mf is None else jnp.maximum(mf, v)
    s = None
    for a, b, q in coef:
        e = jnp.exp(a + b * x + q * x2 - mf)
        s = e if s is None else s + e
    out_ref[...] = mf + jnp.log(s)


def kernel(x, mean_list, scale_list, weight_logits):
    b, z = x.shape
    k = mean_list.shape[-1]
    mean_t = mean_list[0].T
    scale_t = scale_list[0].T

    grid = (b // _TB,)
    return pl.pallas_call(
        _mog_logprob_kernel,
        grid=grid,
        in_specs=[
            pl.BlockSpec((_TB, z), lambda i: (i, 0)),
            pl.BlockSpec((k, z), lambda i: (0, 0)),
            pl.BlockSpec((k, z), lambda i: (0, 0)),
            pl.BlockSpec((1, k), lambda i: (0, 0)),
        ],
        out_specs=pl.BlockSpec((_TB, z), lambda i: (i, 0)),
        out_shape=jax.ShapeDtypeStruct((b, z), x.dtype),
    )(x, mean_t, scale_t, weight_logits)


# speedup vs baseline: 1.6051x; 1.0039x over previous
import math

import jax
import jax.numpy as jnp
from jax.experimental import pallas as pl

_TB = 256


def _mog_logprob_kernel(x_ref, mean_ref, scale_ref, wl_ref, out_ref):
    x = x_ref[...]
    wl = wl_ref[...]
    log_w = wl - jax.nn.logsumexp(wl, axis=-1, keepdims=True)

    k_tot = mean_ref.shape[0]
    half_log_2pi = 0.5 * math.log(2.0 * math.pi)

    x2 = x * x
    coef = []
    for k in range(k_tot):
        sc = jax.nn.softplus(scale_ref[k, :])[None, :]
        mu = mean_ref[k, :][None, :]
        q = -0.5 / (sc * sc)
        b = -2.0 * q * mu
        a = q * mu * mu - jnp.log(sc) - half_log_2pi + log_w[0:1, k : k + 1]
        coef.append((a, b, q))

    mf = None
    for a, b, q in coef:
        v = a + b * x + q * x2
        mf = v if mf is None else jnp.maximum(mf, v)
    s = None
    for a, b, q in coef:
        e = jnp.exp(a + b * x + q * x2 - mf)
        s = e if s is None else s + e
    out_ref[...] = mf + jnp.log(s)


def kernel(x, mean_list, scale_list, weight_logits):
    b, z = x.shape
    k = mean_list.shape[-1]
    mean_t = mean_list[0].T
    scale_t = scale_list[0].T

    grid = (b // _TB,)
    return pl.pallas_call(
        _mog_logprob_kernel,
        grid=grid,
        in_specs=[
            pl.BlockSpec((_TB, z), lambda i: (i, 0)),
            pl.BlockSpec((k, z), lambda i: (0, 0)),
            pl.BlockSpec((k, z), lambda i: (0, 0)),
            pl.BlockSpec((1, k), lambda i: (0, 0)),
        ],
        out_specs=pl.BlockSpec((_TB, z), lambda i: (i, 0)),
        out_shape=jax.ShapeDtypeStruct((b, z), x.dtype),
    )(x, mean_t, scale_t, weight_logits)
